# baseline (speedup 1.0000x reference)
import functools

import jax
import jax.numpy as jnp
from jax import lax
from jax.experimental import pallas as pl
from jax.experimental.pallas import tpu as pltpu
from jax.experimental.pallas import tpu_sc as plsc

D_HALF = 64
D = 2 * D_HALF
_NC = 2
_NS = 16
_NW = _NC * _NS
_SR = 8
_SUB = 80
_LANES = 16


@jax.jit
def _sc_gather_pairs(i_2d, j_2d, ptab):
    b, p = i_2d.shape
    nv = ptab.shape[0]
    n = b * p
    rows_per_w = b // _NW
    stripes = rows_per_w // _SR
    pairs_per_stripe = _SR * p
    subs = pairs_per_stripe // _SUB
    full_vecs = p // _LANES
    tail = p - _LANES
    mesh = plsc.VectorSubcoreMesh(core_axis_name="c", subcore_axis_name="s")

    @functools.partial(
        pl.kernel,
        mesh=mesh,
        out_type=jax.ShapeDtypeStruct((n, D), jnp.float32),
        scratch_types=[
            pltpu.VMEM((_SR, p), jnp.int32),
            pltpu.VMEM((_SR, p), jnp.int32),
            pltpu.VMEM((_SR, p), jnp.int32),
            pltpu.VMEM((_SR, p), jnp.int32),
            pltpu.VMEM((pairs_per_stripe,), jnp.int32),
            pltpu.VMEM((pairs_per_stripe,), jnp.int32),
            pltpu.VMEM((2, _SUB, D), jnp.float32),
            pltpu.VMEM_SHARED((nv, D), jnp.float32),
            pltpu.SemaphoreType.DMA,
            pltpu.SemaphoreType.DMA,
            pltpu.SemaphoreType.DMA,
            pltpu.SemaphoreType.DMA,
            pltpu.SemaphoreType.DMA,
            pltpu.SemaphoreType.DMA,
        ],
    )
    def k(i_hbm, j_hbm, ptab_hbm, out_hbm, iv0, iv1, jv0, jv1, kv0, kv1,
          rows, ptab_sp, sg0, sg1, sw0, sw1, si0, si1):
        iv = (iv0, iv1)
        jv = (jv0, jv1)
        kv = (kv0, kv1)
        sg = (sg0, sg1)
        sw = (sw0, sw1)
        si = (si0, si1)
        wid = lax.axis_index("s") * _NC + lax.axis_index("c")
        wrow0 = wid * rows_per_w

        def idx_start(st, s):
            rb = wrow0 + st * _SR
            pltpu.async_copy(i_hbm.at[pl.ds(rb, _SR)], iv[s], si[s])
            pltpu.async_copy(j_hbm.at[pl.ds(rb, _SR)], jv[s], si[s])

        def idx_wait(st, s):
            rb = wrow0 + st * _SR
            pltpu.make_async_copy(i_hbm.at[pl.ds(rb, _SR)], iv[s], si[s]).wait()
            pltpu.make_async_copy(j_hbm.at[pl.ds(rb, _SR)], jv[s], si[s]).wait()

        def combine(s):
            for r in range(_SR):
                def body(v, c, _r=r):
                    sl = pl.ds(v * _LANES, _LANES)
                    ksl = pl.ds(_r * p + v * _LANES, _LANES)
                    kv[s][ksl] = iv[s][_r, sl] * 100 + jv[s][_r, sl]
                    return c
                lax.fori_loop(0, full_vecs, body, 0)
                tsl = pl.ds(tail, _LANES)
                kv[s][pl.ds(r * p + tail, _LANES)] = (
                    iv[s][r, tsl] * 100 + jv[s][r, tsl]
                )

        def gather_start(s, q, rs):
            pltpu.async_copy(
                ptab_sp.at[kv[s].at[pl.ds(q * _SUB, _SUB)]], rows.at[rs], sg[rs]
            )

        def gather_wait(s, q, rs):
            pltpu.make_async_copy(
                ptab_sp.at[kv[s].at[pl.ds(q * _SUB, _SUB)]], rows.at[rs], sg[rs]
            ).wait()

        def write_start(st, q, rs):
            ob = (wrow0 + st * _SR) * p + q * _SUB
            pltpu.async_copy(rows.at[rs], out_hbm.at[pl.ds(ob, _SUB)], sw[rs])

        def write_wait(st, q, rs):
            ob = (wrow0 + st * _SR) * p + q * _SUB
            pltpu.make_async_copy(
                rows.at[rs], out_hbm.at[pl.ds(ob, _SUB)], sw[rs]
            ).wait()

        @pl.when(lax.axis_index("s") == 0)
        def _():
            pltpu.sync_copy(ptab_hbm, ptab_sp)

        plsc.subcore_barrier()

        idx_start(0, 0)
        idx_wait(0, 0)
        combine(0)
        idx_start(1, 1)
        gather_start(0, 0, 0)

        def outer(g, carry):
            for ss in (0, 1):
                st = 2 * g + ss
                for q in range(subs):
                    rs = q % 2
                    gather_wait(ss, q, rs)
                    write_start(st, q, rs)
                    if q < subs - 1:
                        if q == 0:
                            @pl.when(st >= 1)
                            def _():
                                write_wait(st - 1, subs - 1, 1 - rs)
                        else:
                            write_wait(st, q - 1, 1 - rs)
                        gather_start(ss, q + 1, 1 - rs)
                        if q == 0:
                            @pl.when(st < stripes - 1)
                            def _():
                                idx_wait(st + 1, 1 - ss)
                                combine(1 - ss)

                            @pl.when(st < stripes - 2)
                            def _():
                                idx_start(st + 2, ss)
                    else:
                        @pl.when(st < stripes - 1)
                        def _():
                            write_wait(st, q - 1, 1 - rs)
                            gather_start(1 - ss, 0, 1 - rs)
            return carry

        lax.fori_loop(0, stripes // 2, outer, 0)
        write_wait(stripes - 1, subs - 2, 0)
        write_wait(stripes - 1, subs - 1, 1)

    return k(i_2d, j_2d, ptab)


def kernel(i_indices, j_indices, pe):
    b, p = i_indices.shape
    v = pe.shape[0]
    ptab = jnp.concatenate(
        [jnp.repeat(pe, v, axis=0), jnp.tile(pe, (v, 1))], axis=1
    )
    out = _sc_gather_pairs(i_indices, j_indices, ptab)
    return out.reshape(b, p, D)
